# baseline (speedup 1.0000x reference)
import jax
import jax.numpy as jnp
from jax.experimental import pallas as pl
from jax.experimental.pallas import tpu as pltpu


_ROWS = 2048
_COLS = 8192
_CH = 64
_NBUF = 8


def _make_body(row0, nrows):
    num = nrows // _CH
    ngrp = num // _NBUF

    half = _CH // 2

    def body(w_hbm, t_hbm, o_hbm, w_buf, o_buf, t_v, in_sems, out_sems, t_sem):
        def in_copies(i, b):
            r = row0 + i * _CH
            return (
                pltpu.make_async_copy(
                    w_hbm.at[pl.ds(r, half), :],
                    w_buf.at[b, pl.ds(0, half), :], in_sems.at[b, 0]),
                pltpu.make_async_copy(
                    w_hbm.at[pl.ds(r + half, half), :],
                    w_buf.at[b, pl.ds(half, half), :], in_sems.at[b, 1]),
            )

        def in_copy_start(i, b):
            for cp in in_copies(i, b):
                cp.start()

        def in_copy_wait(i, b):
            for cp in in_copies(i, b):
                cp.wait()

        def out_copies(i, b):
            r = i * _CH
            return (
                pltpu.make_async_copy(
                    o_buf.at[b, pl.ds(0, half), :],
                    o_hbm.at[pl.ds(r, half), :], out_sems.at[b, 0]),
                pltpu.make_async_copy(
                    o_buf.at[b, pl.ds(half, half), :],
                    o_hbm.at[pl.ds(r + half, half), :], out_sems.at[b, 1]),
            )

        def out_copy_start(i, b):
            for cp in out_copies(i, b):
                cp.start()

        def out_copy_wait(i, b):
            for cp in out_copies(i, b):
                cp.wait()

        for b in range(_NBUF):
            in_copy_start(b, b)

        cp = pltpu.make_async_copy(t_hbm.at[pl.ds(row0, nrows), :], t_v, t_sem)
        cp.start()
        cp.wait()
        t_v[...] = jax.nn.sigmoid(t_v[...])

        for i in range(num):
            b = i % _NBUF
            in_copy_wait(i, b)
            if i >= _NBUF:
                out_copy_wait(i - _NBUF, b)
            w = w_buf[b]
            s = t_v[pl.ds(i * _CH, _CH), :]
            o_buf[b] = jnp.maximum(w - s, 0.0) + jnp.minimum(w + s, 0.0)
            out_copy_start(i, b)
            if i + _NBUF < num:
                in_copy_start(i + _NBUF, b)

        for i in range(num - _NBUF, num):
            out_copy_wait(i, i % _NBUF)

    return body


def _masked_rows(weight, threshold, row0, nrows):
    return pl.pallas_call(
        _make_body(row0, nrows),
        in_specs=[
            pl.BlockSpec(memory_space=pl.ANY),
            pl.BlockSpec(memory_space=pl.ANY),
        ],
        out_specs=pl.BlockSpec(memory_space=pl.ANY),
        out_shape=jax.ShapeDtypeStruct((nrows, _COLS), weight.dtype),
        scratch_shapes=[
            pltpu.VMEM((_NBUF, _CH, _COLS), jnp.float32),
            pltpu.VMEM((_NBUF, _CH, _COLS), jnp.float32),
            pltpu.VMEM((nrows, 1), jnp.float32),
            pltpu.SemaphoreType.DMA((_NBUF, 2)),
            pltpu.SemaphoreType.DMA((_NBUF, 2)),
            pltpu.SemaphoreType.DMA,
        ],
    )(weight, threshold)


def kernel(weight, threshold, alpha):
    return _masked_rows(weight, threshold, 0, _ROWS)
